# baseline (speedup 1.0000x reference)
import dataclasses
import functools

import jax
import jax.numpy as jnp
from jax import lax
from jax.experimental import pallas as pl
from jax.experimental.pallas import tpu as pltpu
from jax.experimental.pallas import tpu_sc as plsc

_SAFE1 = 0.1
_SAFE2 = 0.5

_ROWS = 8192
_D = 128
_NCENT = 1000
_NW = 32
_RPW = _ROWS // _NW
_NQ = 4
_Q = _RPW // _NQ
_LANES = 16


def _sc_distances(pooled3, labels2, centroids, delta):
  mesh = plsc.VectorSubcoreMesh(core_axis_name="c", subcore_axis_name="s")
  f32 = jnp.float32
  cp = pltpu.CompilerParams()
  if "needs_layout_passes" in pltpu.CompilerParams.__dataclass_fields__:
    cp = dataclasses.replace(cp, needs_layout_passes=False)

  @functools.partial(
      pl.kernel,
      compiler_params=cp,
      out_type=(
          jax.ShapeDtypeStruct((_ROWS,), f32),
          jax.ShapeDtypeStruct((_ROWS,), f32),
          jax.ShapeDtypeStruct((_ROWS,), f32),
      ),
      mesh=mesh,
      scratch_types=[
          pltpu.VMEM((_NQ, _Q), jnp.int32),
          pltpu.VMEM((_RPW, _D), f32),
          pltpu.VMEM((_RPW, 2, _D), f32),
          pltpu.VMEM((_NCENT,), f32),
          pltpu.VMEM((_RPW,), f32),
          pltpu.VMEM((_RPW,), f32),
          pltpu.VMEM((_RPW,), f32),
          pltpu.SemaphoreType.DMA,
          pltpu.SemaphoreType.DMA,
          pltpu.SemaphoreType.DMA,
          pltpu.SemaphoreType.DMA,
          pltpu.SemaphoreType.DMA,
          pltpu.SemaphoreType.DMA,
          pltpu.SemaphoreType.DMA,
          pltpu.SemaphoreType.DMA,
          pltpu.SemaphoreType.DMA,
          pltpu.SemaphoreType.DMA,
          pltpu.SemaphoreType.DMA,
      ],
  )
  def sc_kernel(pooled_hbm, labels_hbm, cent_hbm, delta_hbm,
                sx_hbm, sn_hbm, dg_hbm,
                lbl_v, c_v, po_v, dtab_v, sx_v, sn_v, dg_v,
                sem_a, sem_c, sem_o,
                sem_p0, sem_p1, sem_p2, sem_p3,
                sem_g0, sem_g1, sem_g2, sem_g3):
    wid = lax.axis_index("s") * 2 + lax.axis_index("c")
    base = wid * _RPW
    psems = [sem_p0, sem_p1, sem_p2, sem_p3]
    gsems = [sem_g0, sem_g1, sem_g2, sem_g3]

    cp_lbl = pltpu.async_copy(
        labels_hbm.at[pl.ds(wid * _NQ, _NQ)], lbl_v, sem_a)
    cp_p = [
        pltpu.async_copy(pooled_hbm.at[pl.ds(base + q * _Q, _Q)],
                         po_v.at[pl.ds(q * _Q, _Q)], psems[q])
        for q in range(_NQ)
    ]
    cp_dt = pltpu.async_copy(delta_hbm, dtab_v, sem_c)
    cp_lbl.wait()

    cp_g = [
        pltpu.async_copy(cent_hbm.at[lbl_v.at[q]],
                         c_v.at[pl.ds(q * _Q, _Q)], gsems[q])
        for q in range(_NQ)
    ]

    cp_dt.wait()
    for t in range(_RPW // _LANES):
      idx = lbl_v[t // 4, pl.ds((t % 4) * _LANES, _LANES)]
      dg_v[pl.ds(t * _LANES, _LANES)] = plsc.load_gather(dtab_v, [idx])
    o3 = pltpu.async_copy(dg_v, dg_hbm.at[pl.ds(base, _RPW)], sem_c)

    lane = lax.iota(jnp.int32, _LANES)

    def quarter_loop(r0):
      @pl.loop(r0 // _LANES, (r0 + _Q) // _LANES)
      def _(t):
        rsx = jnp.zeros((_LANES,), f32)
        rsn = jnp.zeros((_LANES,), f32)
        for l in range(_LANES):
          r = t * _LANES + l
          accx = jnp.zeros((_LANES,), f32)
          accn = jnp.zeros((_LANES,), f32)
          for k in range(_D // _LANES):
            sl = pl.ds(k * _LANES, _LANES)
            c = c_v[r, sl]
            x = po_v[r, 0, sl]
            n = po_v[r, 1, sl]
            tx = x - c
            tn = n - c
            accx = accx + tx * tx
            accn = accn + tn * tn
          rsx = jnp.where(lane == l, jnp.sum(accx), rsx)
          rsn = jnp.where(lane == l, jnp.sum(accn), rsn)
        sx_v[pl.ds(t * _LANES, _LANES)] = rsx
        sn_v[pl.ds(t * _LANES, _LANES)] = rsn

    outs = []
    for q in range(_NQ):
      cp_p[q].wait()
      cp_g[q].wait()
      quarter_loop(q * _Q)
      outs.append(pltpu.async_copy(
          sx_v.at[pl.ds(q * _Q, _Q)], sx_hbm.at[pl.ds(base + q * _Q, _Q)],
          sem_o))
      outs.append(pltpu.async_copy(
          sn_v.at[pl.ds(q * _Q, _Q)], sn_hbm.at[pl.ds(base + q * _Q, _Q)],
          sem_o))
    for o in outs:
      o.wait()
    o3.wait()

  return sc_kernel(pooled3, labels2, centroids, delta)


def _tc_finish(sx, sn, dg, delta):
  f32 = jnp.float32

  def body(sx_ref, sn_ref, dg_ref, delta_ref, loss_ref, dsp_ref):
    euc = jnp.sqrt(sx_ref[...])
    neu = jnp.sqrt(sn_ref[...])
    d = jax.nn.softplus(dg_ref[...])
    pos = jnp.maximum(euc - d, 0.0)
    neg = jnp.maximum(d - euc, 0.0)
    npos = jnp.maximum(neu - (d + _SAFE2), 0.0)
    nneg = jnp.maximum((d - neu) + _SAFE1, 0.0)
    total = (jnp.sum(pos) + jnp.sum(neg)) + (jnp.sum(npos) + jnp.sum(nneg))
    loss_ref[...] = jnp.broadcast_to(total * (1.0 / _ROWS), (1, 1))
    dsp_ref[...] = jax.nn.softplus(delta_ref[...])

  return pl.pallas_call(
      body,
      out_shape=(
          jax.ShapeDtypeStruct((1, 1), f32),
          jax.ShapeDtypeStruct((_NCENT,), f32),
      ),
  )(sx, sn, dg, delta)


def kernel(pooled_output, centroids, labels, delta):
  pooled3 = pooled_output.reshape(_ROWS, 2, _D)
  labels2 = labels.reshape(_ROWS // _Q, _Q)
  sx, sn, dg = _sc_distances(pooled3, labels2, centroids, delta)
  loss2, dsp = _tc_finish(sx, sn, dg, delta)
  return loss2[0, 0], dsp


# speedup vs baseline: 1.0842x; 1.0842x over previous
import dataclasses
import functools

import jax
import jax.numpy as jnp
from jax import lax
from jax.experimental import pallas as pl
from jax.experimental.pallas import tpu as pltpu
from jax.experimental.pallas import tpu_sc as plsc

_SAFE1 = 0.1
_SAFE2 = 0.5

_ROWS = 8192
_D = 128
_NCENT = 1000
_NW = 32
_RPW = _ROWS // _NW
_HALF = _RPW // 2
_LANES = 16


def _sc_distances(pooled3, labels2, centroids, delta):
  mesh = plsc.VectorSubcoreMesh(core_axis_name="c", subcore_axis_name="s")
  f32 = jnp.float32
  cp = pltpu.CompilerParams()
  if "needs_layout_passes" in pltpu.CompilerParams.__dataclass_fields__:
    cp = dataclasses.replace(cp, needs_layout_passes=False)

  @functools.partial(
      pl.kernel,
      compiler_params=cp,
      out_type=(
          jax.ShapeDtypeStruct((_ROWS,), f32),
          jax.ShapeDtypeStruct((_ROWS,), f32),
          jax.ShapeDtypeStruct((_ROWS,), f32),
      ),
      mesh=mesh,
      scratch_types=[
          pltpu.VMEM((2, 128), jnp.int32),
          pltpu.VMEM((_RPW, _D), f32),
          pltpu.VMEM((_RPW, 2, _D), f32),
          pltpu.VMEM((_NCENT,), f32),
          pltpu.VMEM((_RPW,), f32),
          pltpu.VMEM((_RPW,), f32),
          pltpu.VMEM((_RPW,), f32),
          pltpu.SemaphoreType.DMA,
          pltpu.SemaphoreType.DMA,
          pltpu.SemaphoreType.DMA,
          pltpu.SemaphoreType.DMA,
          pltpu.SemaphoreType.DMA,
          pltpu.SemaphoreType.DMA,
      ],
  )
  def sc_kernel(pooled_hbm, labels_hbm, cent_hbm, delta_hbm,
                sx_hbm, sn_hbm, dg_hbm,
                lbl_v, c_v, po_v, dtab_v, sx_v, sn_v, dg_v,
                sem_a, sem_c, sem_p0, sem_p1, sem_g0, sem_g1):
    wid = lax.axis_index("s") * 2 + lax.axis_index("c")
    base = wid * _RPW

    cp_lbl = pltpu.async_copy(labels_hbm.at[pl.ds(wid * 2, 2)], lbl_v, sem_a)
    cp_p0 = pltpu.async_copy(
        pooled_hbm.at[pl.ds(base, _HALF)], po_v.at[pl.ds(0, _HALF)], sem_p0)
    cp_p1 = pltpu.async_copy(
        pooled_hbm.at[pl.ds(base + _HALF, _HALF)],
        po_v.at[pl.ds(_HALF, _HALF)], sem_p1)
    cp_dt = pltpu.async_copy(delta_hbm, dtab_v, sem_c)
    cp_lbl.wait()

    cp_g0 = pltpu.async_copy(
        cent_hbm.at[lbl_v.at[0]], c_v.at[pl.ds(0, _HALF)], sem_g0)
    cp_g1 = pltpu.async_copy(
        cent_hbm.at[lbl_v.at[1]], c_v.at[pl.ds(_HALF, _HALF)], sem_g1)

    cp_dt.wait()
    for t in range(_RPW // _LANES):
      idx = lbl_v[t // 8, pl.ds((t % 8) * _LANES, _LANES)]
      dg_v[pl.ds(t * _LANES, _LANES)] = plsc.load_gather(dtab_v, [idx])
    o3 = pltpu.async_copy(dg_v, dg_hbm.at[pl.ds(base, _RPW)], sem_c)

    lane = lax.iota(jnp.int32, _LANES)

    def half_loop(r0):
      @pl.loop(r0 // _LANES, (r0 + _HALF) // _LANES)
      def _(t):
        rsx = jnp.zeros((_LANES,), f32)
        rsn = jnp.zeros((_LANES,), f32)
        for l in range(_LANES):
          r = t * _LANES + l
          accx = jnp.zeros((_LANES,), f32)
          accn = jnp.zeros((_LANES,), f32)
          for k in range(_D // _LANES):
            sl = pl.ds(k * _LANES, _LANES)
            c = c_v[r, sl]
            x = po_v[r, 0, sl]
            n = po_v[r, 1, sl]
            tx = x - c
            tn = n - c
            accx = accx + tx * tx
            accn = accn + tn * tn
          rsx = jnp.where(lane == l, jnp.sum(accx), rsx)
          rsn = jnp.where(lane == l, jnp.sum(accn), rsn)
        sx_v[pl.ds(t * _LANES, _LANES)] = rsx
        sn_v[pl.ds(t * _LANES, _LANES)] = rsn

    cp_p0.wait()
    cp_g0.wait()
    half_loop(0)
    o1a = pltpu.async_copy(
        sx_v.at[pl.ds(0, _HALF)], sx_hbm.at[pl.ds(base, _HALF)], sem_p0)
    o2a = pltpu.async_copy(
        sn_v.at[pl.ds(0, _HALF)], sn_hbm.at[pl.ds(base, _HALF)], sem_g0)
    cp_p1.wait()
    cp_g1.wait()
    half_loop(_HALF)
    o1b = pltpu.async_copy(
        sx_v.at[pl.ds(_HALF, _HALF)], sx_hbm.at[pl.ds(base + _HALF, _HALF)],
        sem_p1)
    o2b = pltpu.async_copy(
        sn_v.at[pl.ds(_HALF, _HALF)], sn_hbm.at[pl.ds(base + _HALF, _HALF)],
        sem_g1)
    o1a.wait()
    o2a.wait()
    o1b.wait()
    o2b.wait()
    o3.wait()

  return sc_kernel(pooled3, labels2, centroids, delta)


def _tc_finish(sx, sn, dg, delta):
  f32 = jnp.float32

  def body(sx_ref, sn_ref, dg_ref, delta_ref, loss_ref, dsp_ref):
    euc = jnp.sqrt(sx_ref[...])
    neu = jnp.sqrt(sn_ref[...])
    d = jax.nn.softplus(dg_ref[...])
    pos = jnp.maximum(euc - d, 0.0)
    neg = jnp.maximum(d - euc, 0.0)
    npos = jnp.maximum(neu - (d + _SAFE2), 0.0)
    nneg = jnp.maximum((d - neu) + _SAFE1, 0.0)
    total = (jnp.sum(pos) + jnp.sum(neg)) + (jnp.sum(npos) + jnp.sum(nneg))
    loss_ref[...] = jnp.broadcast_to(total * (1.0 / _ROWS), (1, 1))
    dsp_ref[...] = jax.nn.softplus(delta_ref[...])

  return pl.pallas_call(
      body,
      out_shape=(
          jax.ShapeDtypeStruct((1, 1), f32),
          jax.ShapeDtypeStruct((_NCENT,), f32),
      ),
  )(sx, sn, dg, delta)


def kernel(pooled_output, centroids, labels, delta):
  pooled3 = pooled_output.reshape(_ROWS, 2, _D)
  labels2 = labels.reshape(_ROWS // 128, 128)
  sx, sn, dg = _sc_distances(pooled3, labels2, centroids, delta)
  loss2, dsp = _tc_finish(sx, sn, dg, delta)
  return loss2[0, 0], dsp
